# Initial kernel scaffold; baseline (speedup 1.0000x reference)
import jax
import jax.numpy as jnp
from jax.experimental import pallas as pl


def kernel(inputs, code_snippet_id, data, w):
    raise NotImplementedError("write your pallas kernel here")


# speedup vs baseline: 1.1781x; 1.1781x over previous
import functools

import jax
import jax.numpy as jnp
from jax import lax
from jax.experimental import pallas as pl
from jax.experimental.pallas import tpu as pltpu
from jax.experimental.pallas import tpu_sc as plsc

_E = 768
_P = 166
_B = 16
_S = 1126
_W = 5
_CH = 64
_NCH = 18
_TAIL = _S - (_NCH - 1) * _CH
_SPAD = _NCH * _CH
_NW = 32
_PERW = (_B * _NCH) // _NW


def _comb_kernel(csid_ref, data_ref, w_ref, out_ref):
    del csid_ref
    x = data_ref[0]
    snip = x[:, :_E]
    wt = w_ref[...]
    shifted = (
        jnp.concatenate([snip[:1], snip[:1], snip[:-2]], axis=0),
        jnp.concatenate([snip[:1], snip[:-1]], axis=0),
        snip,
        jnp.concatenate([snip[1:], snip[-1:]], axis=0),
        jnp.concatenate([snip[2:], snip[-1:], snip[-1:]], axis=0),
    )
    acc = shifted[0] * wt[0:1, :]
    for k in range(1, _W):
        acc = acc + shifted[k] * wt[k:k + 1, :]
    out_ref[0] = acc


def _build_comb(csid, data_r, w):
    grid_spec = pltpu.PrefetchScalarGridSpec(
        num_scalar_prefetch=1,
        grid=(_B,),
        in_specs=[
            pl.BlockSpec((1, _P, 2 * _E), lambda b, csid_ref: (csid_ref[b], 0, 0)),
            pl.BlockSpec((_W, _E), lambda b, csid_ref: (0, 0)),
        ],
        out_specs=pl.BlockSpec((1, _P, _E), lambda b, csid_ref: (b, 0, 0)),
    )
    return pl.pallas_call(
        _comb_kernel,
        grid_spec=grid_spec,
        out_shape=jax.ShapeDtypeStruct((_B, _P, _E), jnp.float32),
    )(csid, data_r, w)


def _sc_gather_body(comb_hbm, inp_hbm, out_hbm, idx_v, gidx_v, rows_v, sem):
    wid = lax.axis_index("s") * 2 + lax.axis_index("c")
    b = wid // 2
    c0 = (wid % 2) * _PERW
    for i in range(_PERW):
        c = c0 + i
        s0 = c * _CH
        pltpu.sync_copy(inp_hbm.at[pl.ds(b * _SPAD + s0, _CH)], idx_v)
        for j in range(_CH // 16):
            v = idx_v[pl.ds(j * 16, 16)]
            v = jnp.clip(v, 0, _P - 1) + b * _P
            gidx_v[pl.ds(j * 16, 16)] = v
        pltpu.async_copy(comb_hbm.at[gidx_v], rows_v, sem).wait()

        @pl.when(c != _NCH - 1)
        def _():
            pltpu.sync_copy(rows_v, out_hbm.at[pl.ds(b * _S + s0, _CH)])

        @pl.when(c == _NCH - 1)
        def _():
            pltpu.sync_copy(rows_v.at[pl.ds(0, _TAIL)],
                            out_hbm.at[pl.ds(b * _S + s0, _TAIL)])


_sc_gather_cache = []


def _sc_gather():
    if not _sc_gather_cache:
        _sc_gather_cache.append(functools.partial(
            pl.kernel,
            out_type=jax.ShapeDtypeStruct((_B * _S, _E), jnp.float32),
            mesh=plsc.VectorSubcoreMesh(core_axis_name="c", subcore_axis_name="s"),
            compiler_params=pltpu.CompilerParams(use_tc_tiling_on_sc=False),
            scratch_types=[
                pltpu.VMEM((_CH,), jnp.int32),
                pltpu.VMEM((_CH,), jnp.int32),
                pltpu.VMEM((_CH, _E), jnp.float32),
                pltpu.SemaphoreType.DMA,
            ],
        )(_sc_gather_body))
    return _sc_gather_cache[0]


def kernel(inputs, code_snippet_id, data, w):
    inputs = inputs.astype(jnp.int32)
    csid = code_snippet_id.astype(jnp.int32).reshape(_B)
    data_r = data.reshape(data.shape[0], _P, 2 * _E)
    comb = _build_comb(csid, data_r, w.astype(jnp.float32))
    comb2 = comb.reshape(_B * _P, _E)
    inp_flat = jnp.pad(inputs, ((0, 0), (0, _SPAD - _S))).reshape(_B * _SPAD)
    out = _sc_gather()(comb2, inp_flat)
    return out.reshape(_B, _S, _E)


# speedup vs baseline: 1.2569x; 1.0669x over previous
import functools

import jax
import jax.numpy as jnp
from jax import lax
from jax.experimental import pallas as pl
from jax.experimental.pallas import tpu as pltpu
from jax.experimental.pallas import tpu_sc as plsc

_E = 768
_P = 166
_PP = 168
_B = 16
_S = 1126
_W = 5
_CH = 64
_NCH = 18
_SP = 1128
_TAILW = _SP - (_NCH - 1) * _CH
_TAIL = _S - (_NCH - 1) * _CH
_NW = 32
_PERW = (_B * _NCH) // _NW


def _comb_kernel(csid_ref, data_ref, w_ref, out_ref):
    del csid_ref
    x = data_ref[0]
    snip = x[:, :_E]
    wt = w_ref[...]
    shifted = (
        jnp.concatenate([snip[:1], snip[:1], snip[:-2]], axis=0),
        jnp.concatenate([snip[:1], snip[:-1]], axis=0),
        snip,
        jnp.concatenate([snip[1:], snip[-1:]], axis=0),
        jnp.concatenate([snip[2:], snip[-1:], snip[-1:]], axis=0),
    )
    acc = shifted[0] * wt[0:1, :]
    for k in range(1, _W):
        acc = acc + shifted[k] * wt[k:k + 1, :]
    out_ref[0] = jnp.concatenate([acc, acc[-2:]], axis=0)


def _build_comb(csid, data_r, w):
    grid_spec = pltpu.PrefetchScalarGridSpec(
        num_scalar_prefetch=1,
        grid=(_B,),
        in_specs=[
            pl.BlockSpec((1, _P, 2 * _E), lambda b, csid_ref: (csid_ref[b], 0, 0)),
            pl.BlockSpec((_W, _E), lambda b, csid_ref: (0, 0)),
        ],
        out_specs=pl.BlockSpec((1, _PP, _E), lambda b, csid_ref: (b, 0, 0)),
    )
    return pl.pallas_call(
        _comb_kernel,
        grid_spec=grid_spec,
        out_shape=jax.ShapeDtypeStruct((_B, _PP, _E), jnp.float32),
    )(csid, data_r, w)


def _sc_gather_body(comb_hbm, inpc_hbm, out_hbm, idx_v, gidx_v,
                    rows0, rows1, gsem, wsem0, wsem1):
    cid = lax.axis_index("c")
    sid = lax.axis_index("s")
    wid = cid * 16 + sid
    b = wid // 2
    c0 = (wid % 2) * _PERW
    bufs = (rows0, rows1)
    wsems = (wsem0, wsem1)
    write_cps = []
    for i in range(_PERW):
        c = c0 + i
        s0 = c * _CH
        buf = bufs[i % 2]
        if i >= 2:
            write_cps[i - 2].wait()
        pltpu.sync_copy(inpc_hbm.at[b * _NCH + c], idx_v)
        for j in range(_CH // 16):
            v = idx_v[0, pl.ds(j * 16, 16)]
            v = jnp.clip(v, 0, _P - 1) + b * _PP
            gidx_v[pl.ds(j * 16, 16)] = v
        pltpu.async_copy(comb_hbm.at[gidx_v], buf, gsem).wait()
        if i < _PERW - 1:
            cp = pltpu.make_async_copy(
                buf, out_hbm.at[b, pl.ds(s0, _CH)], wsems[i % 2])
            cp.start()
            write_cps.append(cp)
        else:
            @pl.when(c == _NCH - 1)
            def _():
                pltpu.sync_copy(buf.at[pl.ds(0, _TAILW)],
                                out_hbm.at[b, pl.ds(s0, _TAILW)])

            @pl.when(c != _NCH - 1)
            def _():
                pltpu.sync_copy(buf, out_hbm.at[b, pl.ds(s0, _CH)])
    write_cps[-1].wait()


_sc_gather_cache = []


def _sc_gather():
    if not _sc_gather_cache:
        _sc_gather_cache.append(functools.partial(
            pl.kernel,
            out_type=jax.ShapeDtypeStruct((_B, _SP, _E), jnp.float32),
            mesh=plsc.VectorSubcoreMesh(core_axis_name="c", subcore_axis_name="s"),
            scratch_types=[
                pltpu.VMEM((1, _CH), jnp.int32),
                pltpu.VMEM((_CH,), jnp.int32),
                pltpu.VMEM((_CH, _E), jnp.float32),
                pltpu.VMEM((_CH, _E), jnp.float32),
                pltpu.SemaphoreType.DMA,
                pltpu.SemaphoreType.DMA,
                pltpu.SemaphoreType.DMA,
            ],
        )(_sc_gather_body))
    return _sc_gather_cache[0]


def kernel(inputs, code_snippet_id, data, w):
    inputs = inputs.astype(jnp.int32)
    csid = code_snippet_id.astype(jnp.int32).reshape(_B)
    data_r = data.reshape(data.shape[0], _P, 2 * _E)
    comb = _build_comb(csid, data_r, w.astype(jnp.float32))
    comb2 = comb.reshape(_B * _PP, _E)
    inp_chunks = jnp.concatenate(
        [inputs[:, :(_NCH - 1) * _CH].reshape(_B, _NCH - 1, _CH),
         jnp.pad(inputs[:, (_NCH - 1) * _CH:], ((0, 0), (0, _CH - _TAIL))
                 ).reshape(_B, 1, _CH)], axis=1,
    ).reshape(_B * _NCH, 1, _CH)
    out = _sc_gather()(comb2, inp_chunks)
    return out[:, :_S, :]
